# Initial kernel scaffold; baseline (speedup 1.0000x reference)
import jax
import jax.numpy as jnp
from jax.experimental import pallas as pl


def kernel(inputs, init_state, support0, support1, Wg0, bg0, Wc0, bc0, Wg1, bg1, Wc1, bc1, Wp, bp):
    raise NotImplementedError("write your pallas kernel here")


# speedup vs baseline: 3.7246x; 3.7246x over previous
import numpy as np
import jax
import jax.numpy as jnp
from jax.experimental import pallas as pl
from jax.experimental.pallas import tpu as pltpu

N_NODE = 325
IN_DIM = 2
HID = 64
ORD = 2
BATCH_N = 64
SEQ_N = 12
KDIFF = 1 + 2 * ORD

B_CHUNK = 16

_TF = (np.random.RandomState(42).rand(SEQ_N - 1) < 0.5).tolist()

_BF = jnp.bfloat16


def _mm(a, b):
    return jax.lax.dot_general(a.astype(_BF), b.astype(_BF),
                               (((a.ndim - 1,), (0,)), ((), ())),
                               preferred_element_type=jnp.float32)


def _decoder_body(xseq_ref, tf_ref, s0_ref, s1_ref, a0_ref, a1_ref,
                  wg0_ref, wc0_ref, wg1_ref, wc1_ref,
                  bg0_ref, bc0_ref, bg1_ref, bc1_ref,
                  wp_ref, bp_ref, out_ref):
    a0 = a0_ref[...]
    a1 = a1_ref[...]
    row = jax.lax.broadcasted_iota(jnp.int32, (N_NODE, N_NODE), 0)
    col = jax.lax.broadcasted_iota(jnp.int32, (N_NODE, N_NODE), 1)
    eye = (row == col).astype(jnp.float32)
    a0c = 2.0 * jax.lax.dot_general(a0, a0, (((1,), (0,)), ((), ())),
                                    preferred_element_type=jnp.float32) - eye
    a1c = 2.0 * jax.lax.dot_general(a1, a1, (((1,), (0,)), ((), ())),
                                    preferred_element_type=jnp.float32) - eye
    amats = (a0.astype(_BF), a0c.astype(_BF), a1.astype(_BF), a1c.astype(_BF))

    def prop(ab, v):
        return jax.lax.dot_general(ab, v.astype(_BF), (((1,), (0,)), ((), ())),
                                   preferred_element_type=jnp.float32)

    def gconv(cat, w_ref, bias, width):
        v = _mm(cat, w_ref[...])
        acc = bias + v[..., :width]
        for k in range(1, KDIFF):
            acc = acc + prop(amats[k - 1], v[..., k * width:(k + 1) * width])
        return acc

    def cell(x64, h, wg_ref, wc_ref, bg, bc):
        cat = jnp.concatenate([x64, h], axis=-1)
        g = jax.nn.sigmoid(gconv(cat, wg_ref, bg, 2 * HID))
        r = g[..., :HID]
        u = g[..., HID:]
        cat2 = jnp.concatenate([x64, r * h], axis=-1)
        c = jnp.tanh(gconv(cat2, wc_ref, bc, HID))
        return u * h + (1.0 - u) * c

    bg0 = bg0_ref[...].reshape(1, 1, 2 * HID)
    bc0 = bc0_ref[...].reshape(1, 1, HID)
    bg1 = bg1_ref[...].reshape(1, 1, 2 * HID)
    bc1 = bc1_ref[...].reshape(1, 1, HID)
    bp = bp_ref[...].reshape(1, 1, IN_DIM)
    wp = wp_ref[...]

    def xin64(t):
        x = xseq_ref[t, 0].reshape(N_NODE, B_CHUNK, IN_DIM)
        return jnp.pad(x, ((0, 0), (0, 0), (0, HID - IN_DIM)))

    def step(t, carry):
        cur64, s0, s1 = carry
        s0 = cell(cur64, s0, wg0_ref, wc0_ref, bg0, bc0)
        s1 = cell(s0, s1, wg1_ref, wc1_ref, bg1, bc1)
        proj = _mm(s1, wp) + bp
        out_ref[t, 0] = proj.reshape(N_NODE, B_CHUNK * IN_DIM)
        tfv = tf_ref[t].astype(jnp.float32)
        nxt = tfv * xin64(t + 1) + (1.0 - tfv) * jnp.pad(
            proj, ((0, 0), (0, 0), (0, HID - IN_DIM)))
        return nxt, s0, s1

    carry = (xin64(0), s0_ref[...], s1_ref[...])
    jax.lax.fori_loop(0, SEQ_N - 1, step, carry)


def kernel(inputs, init_state, support0, support1,
           Wg0, bg0, Wc0, bc0, Wg1, bg1, Wc1, bc1, Wp, bp):
    f32 = jnp.float32
    nchunk = BATCH_N // B_CHUNK
    xseq = jnp.transpose(inputs, (1, 2, 0, 3)).reshape(
        SEQ_N, N_NODE, nchunk, B_CHUNK * IN_DIM).transpose(0, 2, 1, 3)
    st = jnp.transpose(init_state, (0, 2, 1, 3))

    def prep(W, out_w, cin_x):
        w = W.reshape(KDIFF, cin_x + HID, out_w)
        wx = jnp.pad(w[:, :cin_x, :], ((0, 0), (0, HID - cin_x), (0, 0)))
        w = jnp.concatenate([wx, w[:, cin_x:, :]], axis=1)
        return jnp.transpose(w, (1, 0, 2)).reshape(2 * HID, KDIFF * out_w)

    tf_arr = jnp.array([1.0 if v else 0.0 for v in _TF], dtype=f32)

    args = (
        xseq, tf_arr, st[0], st[1], support0, support1,
        prep(Wg0, 2 * HID, IN_DIM), prep(Wc0, HID, IN_DIM),
        prep(Wg1, 2 * HID, HID), prep(Wc1, HID, HID),
        bg0.reshape(1, 2 * HID), bc0.reshape(1, HID),
        bg1.reshape(1, 2 * HID), bc1.reshape(1, HID),
        Wp, bp.reshape(1, IN_DIM),
    )

    fixed = lambda *shape: pl.BlockSpec(shape, lambda i: (0,) * len(shape))
    in_specs = [
        pl.BlockSpec((SEQ_N, 1, N_NODE, B_CHUNK * IN_DIM),
                     lambda i: (0, i, 0, 0)),
        pl.BlockSpec(memory_space=pltpu.SMEM),
        pl.BlockSpec((N_NODE, B_CHUNK, HID), lambda i: (0, i, 0)),
        pl.BlockSpec((N_NODE, B_CHUNK, HID), lambda i: (0, i, 0)),
        fixed(N_NODE, N_NODE), fixed(N_NODE, N_NODE),
        fixed(2 * HID, KDIFF * 2 * HID), fixed(2 * HID, KDIFF * HID),
        fixed(2 * HID, KDIFF * 2 * HID), fixed(2 * HID, KDIFF * HID),
        fixed(1, 2 * HID), fixed(1, HID), fixed(1, 2 * HID), fixed(1, HID),
        fixed(HID, IN_DIM), fixed(1, IN_DIM),
    ]
    out = pl.pallas_call(
        _decoder_body,
        grid=(nchunk,),
        in_specs=in_specs,
        out_specs=pl.BlockSpec((SEQ_N - 1, 1, N_NODE, B_CHUNK * IN_DIM),
                               lambda i: (0, i, 0, 0)),
        out_shape=jax.ShapeDtypeStruct(
            (SEQ_N - 1, nchunk, N_NODE, B_CHUNK * IN_DIM), f32),
        compiler_params=pltpu.CompilerParams(
            dimension_semantics=("arbitrary",),
        ),
    )(*[a.astype(f32) for a in args])
    out = out.transpose(0, 2, 1, 3).reshape(SEQ_N - 1, N_NODE, BATCH_N, IN_DIM)
    return jnp.transpose(out, (2, 0, 1, 3))
